# baseline (speedup 1.0000x reference)
import jax
import jax.numpy as jnp
from jax.experimental import pallas as pl
from jax.experimental.pallas import tpu as pltpu

HBLK = 2


def _scores_body(x_ref, widx_ref, focus_ref, sc_ref):
    b = pl.program_id(0)
    nb = pl.num_programs(0)
    x = x_ref[...]
    Tf = x.shape[1]
    m = jnp.max(x, axis=2)
    for i in range(HBLK):
        sc_ref[b * HBLK + i] = jnp.sum(m[i])

    @pl.when(b == nb - 1)
    def _():
        nh = HBLK * nb

        def step(j, carry):
            best, idx = carry
            v = sc_ref[j]
            take = v > best
            return jnp.where(take, v, best), jnp.where(take, j, idx)

        best, idx = jax.lax.fori_loop(0, nh, step, (jnp.float32(-1.0), jnp.int32(0)))
        widx_ref[0] = idx
        focus_ref[0] = best / Tf


def _winner_body(widx_ref, x_ref, dur_ref):
    x = x_ref[0]
    Tf, Tt = x.shape
    rowmax = jnp.max(x, axis=1, keepdims=True)
    ids = jax.lax.broadcasted_iota(jnp.int32, (Tf, Tt), 1)
    am = jnp.min(jnp.where(x == rowmax, ids, Tt), axis=1, keepdims=True)
    onehot = (am == ids).astype(jnp.int32)
    dur_ref[...] = jnp.sum(onehot, axis=0, keepdims=True)


def kernel(att_ws):
    L, H, Tf, Tt = att_ws.shape
    NH = L * H
    flat = att_ws.reshape(NH, Tf, Tt)
    widx, focus = pl.pallas_call(
        _scores_body,
        grid=(NH // HBLK,),
        in_specs=[pl.BlockSpec((HBLK, Tf, Tt), lambda b: (b, 0, 0))],
        out_specs=[
            pl.BlockSpec(memory_space=pltpu.SMEM),
            pl.BlockSpec(memory_space=pltpu.SMEM),
        ],
        out_shape=[
            jax.ShapeDtypeStruct((1,), jnp.int32),
            jax.ShapeDtypeStruct((1,), jnp.float32),
        ],
        scratch_shapes=[pltpu.SMEM((NH,), jnp.float32)],
    )(flat)
    dur = pl.pallas_call(
        _winner_body,
        grid_spec=pltpu.PrefetchScalarGridSpec(
            num_scalar_prefetch=1,
            grid=(1,),
            in_specs=[pl.BlockSpec((1, Tf, Tt), lambda g, w: (w[0], 0, 0))],
            out_specs=pl.BlockSpec((1, Tt), lambda g, w: (0, 0)),
        ),
        out_shape=jax.ShapeDtypeStruct((1, Tt), jnp.int32),
    )(widx, flat)
    durations = dur[0].astype(jnp.int64)
    focus_rate = focus[0]
    return durations, focus_rate
